# Initial kernel scaffold; baseline (speedup 1.0000x reference)
import jax
import jax.numpy as jnp
from jax.experimental import pallas as pl


def kernel(x, mask, W_embed, rotations, W_cls, b_cls, Wqk_0, Wv_0, Wo_0, W1_0, b1_0, W2_0, b2_0, g1_0, beta1_0, g2_0, beta2_0, Wqk_1, Wv_1, Wo_1, W1_1, b1_1, W2_1, b2_1, g1_1, beta1_1, g2_1, beta2_1):
    raise NotImplementedError("write your pallas kernel here")


# speedup vs baseline: 8.3679x; 8.3679x over previous
import functools
import math

import jax
import jax.numpy as jnp
import numpy as np
from jax import lax
from jax.experimental import pallas as pl
from jax.experimental.pallas import tpu as pltpu
from jax.experimental.pallas import tpu_sc as plsc

B, S, ENC_IN = 4, 2048, 768
D_MODEL, D_FF, N_HEADS, E_LAYERS = 1024, 2048, 16, 2
BUCKET = 64
N_HASHES = 4
NUM_CLASS = 256
D_HEAD = 64
PK = 2 * D_HEAD
R_TOT = N_HASHES * B * N_HEADS
NROW = B * N_HEADS


def _embed_body(x_ref, w_ref, o_ref):
    o_ref[...] = jnp.dot(x_ref[...], w_ref[...], preferred_element_type=jnp.float32)


def _embed(x, W_embed):
    x2 = x.reshape(B * S, ENC_IN)
    T = 512
    out = pl.pallas_call(
        _embed_body,
        grid=(B * S // T,),
        in_specs=[pl.BlockSpec((T, ENC_IN), lambda i: (i, 0)),
                  pl.BlockSpec((ENC_IN, D_MODEL), lambda i: (0, 0))],
        out_specs=pl.BlockSpec((T, D_MODEL), lambda i: (i, 0)),
        out_shape=jax.ShapeDtypeStruct((B * S, D_MODEL), jnp.float32),
    )(x2, W_embed)
    return out


_PT = 256


def _proj_body(h_ref, wqk_ref, wv_ref, rot_ref, qkv_ref, bk_ref):
    h = h_ref[...]
    qk = jnp.dot(h, wqk_ref[...], preferred_element_type=jnp.float32)
    v = jnp.dot(h, wv_ref[...], preferred_element_type=jnp.float32)
    pieces = []
    for hd in range(N_HEADS):
        qh = qk[:, hd * D_HEAD:(hd + 1) * D_HEAD]
        pieces.append(qh)
        pieces.append(v[:, hd * D_HEAD:(hd + 1) * D_HEAD])
        nrm = jnp.sqrt(jnp.sum(qh * qh, axis=-1, keepdims=True))
        qn = qh / (nrm + 1e-6)
        proj = jnp.dot(qn, rot_ref[...], preferred_element_type=jnp.float32)
        for hh in range(N_HASHES):
            p = proj[:, hh * BUCKET:(hh + 1) * BUCKET]
            m1 = jnp.max(p, axis=-1)
            a1 = jnp.argmax(p, axis=-1).astype(jnp.int32)
            m2 = jnp.max(-p, axis=-1)
            a2 = jnp.argmax(-p, axis=-1).astype(jnp.int32)
            bk_ref[hh, 0, hd, :] = jnp.where(m1 >= m2, a1, BUCKET + a2)
    qkv_ref[...] = jnp.concatenate(pieces, axis=-1)


def _proj_hash(h2, Wqk, Wv, rot_cat):
    TPB = S // _PT
    qkv, buckets = pl.pallas_call(
        _proj_body,
        grid=(B * S // _PT,),
        in_specs=[pl.BlockSpec((_PT, D_MODEL), lambda i: (i, 0)),
                  pl.BlockSpec((D_MODEL, D_MODEL), lambda i: (0, 0)),
                  pl.BlockSpec((D_MODEL, D_MODEL), lambda i: (0, 0)),
                  pl.BlockSpec((D_HEAD, N_HASHES * BUCKET), lambda i: (0, 0))],
        out_specs=[pl.BlockSpec((_PT, N_HEADS * PK), lambda i: (i, 0)),
                   pl.BlockSpec((N_HASHES, 1, N_HEADS, _PT),
                                lambda i: (0, i // TPB, 0, i % TPB))],
        out_shape=[jax.ShapeDtypeStruct((B * S, N_HEADS * PK), jnp.float32),
                   jax.ShapeDtypeStruct((N_HASHES, B, N_HEADS, S), jnp.int32)],
    )(h2, Wqk, Wv, rot_cat)
    return qkv, buckets


_NW = 32
_RPW = R_TOT // _NW
_GC = 16
_GB = S // _GC

_SC_PARAMS = pltpu.CompilerParams(needs_layout_passes=False)


def _sc_mesh():
    return plsc.VectorSubcoreMesh(core_axis_name="c", subcore_axis_name="s")


def _sortgather_body(bkt_hbm, qkv_hbm, inv_hbm, qkvs_hbm,
                     bkt_v, hist_v, lrank_v, offs_v, ord_v, inv_v, gidx_v,
                     b0, b1, sem):
    wid = lax.axis_index("s") * 2 + lax.axis_index("c")
    iota = lax.iota(jnp.int32, 16)

    def row_body(k, _):
        row = wid * _RPW + k
        r2 = lax.rem(row, NROW)
        b = lax.div(r2, N_HEADS)
        hd = lax.rem(r2, N_HEADS)
        gcst = b * (S * N_HEADS) + hd
        pltpu.sync_copy(bkt_hbm.at[row], bkt_v)

        def zero(i, _):
            hist_v[pl.ds(i * 16, 16)] = jnp.zeros((16,), jnp.int32)
            return 0
        lax.fori_loop(0, 128, zero, 0)

        def pa(i, _):
            idx = iota * 128 + i
            bk = plsc.load_gather(bkt_v, [idx])
            hidx = bk * 16 + iota
            cnt = plsc.load_gather(hist_v, [hidx])
            plsc.store_scatter(lrank_v, [idx], cnt)
            plsc.store_scatter(hist_v, [hidx], cnt + 1)
            return 0
        lax.fori_loop(0, 128, pa, 0)

        def pb(i, run):
            hvec = hist_v[pl.ds(i * 16, 16)]
            cs = plsc.cumsum(hvec)
            hist_v[pl.ds(i * 16, 16)] = cs - hvec
            tot = jnp.sum(hvec)
            plsc.store_scatter(offs_v, [jnp.full((16,), i, jnp.int32)],
                               jnp.full((16,), run, jnp.int32),
                               mask=iota == 0)
            return run + tot
        lax.fori_loop(0, 128, pb, jnp.int32(0))

        def pc(i, _):
            idx = iota * 128 + i
            bk = plsc.load_gather(bkt_v, [idx])
            lp = plsc.load_gather(hist_v, [bk * 16 + iota])
            og = plsc.load_gather(offs_v, [bk])
            lr = plsc.load_gather(lrank_v, [idx])
            rank = og + lp + lr
            plsc.store_scatter(inv_v, [idx], rank)
            plsc.store_scatter(ord_v, [rank], idx)
            return 0
        lax.fori_loop(0, 128, pc, 0)
        pltpu.sync_copy(inv_v, inv_hbm.at[row])

        for c in range(_GC):
            def gb(j, _, c=c):
                ov = ord_v[pl.ds(c * _GB + j * 16, 16)]
                gidx_v[c, pl.ds(j * 16, 16)] = ov * N_HEADS + gcst
                return 0
            lax.fori_loop(0, _GB // 16, gb, 0)

        base = row * S
        bufs = (b0, b1)
        waits = [None, None]
        for c in range(_GC):
            w = pltpu.async_copy(qkv_hbm.at[gidx_v.at[c]], bufs[c % 2], sem)
            if c > 0:
                waits[(c - 1) % 2].wait()
                pltpu.sync_copy(bufs[(c - 1) % 2],
                                qkvs_hbm.at[pl.ds(base + (c - 1) * _GB, _GB)])
            waits[c % 2] = w
        waits[(_GC - 1) % 2].wait()
        pltpu.sync_copy(bufs[(_GC - 1) % 2],
                        qkvs_hbm.at[pl.ds(base + (_GC - 1) * _GB, _GB)])
        return 0

    lax.fori_loop(0, _RPW, row_body, 0)


def _sort_gather(buckets, qkv_rows):
    f = pl.kernel(
        _sortgather_body,
        out_type=[jax.ShapeDtypeStruct((R_TOT, S), jnp.int32),
                  jax.ShapeDtypeStruct((R_TOT * S, PK), jnp.float32)],
        mesh=_sc_mesh(),
        compiler_params=_SC_PARAMS,
        scratch_types=[
            pltpu.VMEM((S,), jnp.int32),
            pltpu.VMEM((S,), jnp.int32),
            pltpu.VMEM((S,), jnp.int32),
            pltpu.VMEM((128,), jnp.int32),
            pltpu.VMEM((S,), jnp.int32),
            pltpu.VMEM((S,), jnp.int32),
            pltpu.VMEM((_GC, _GB), jnp.int32),
            pltpu.VMEM((_GB, PK), jnp.float32),
            pltpu.VMEM((_GB, PK), jnp.float32),
            pltpu.SemaphoreType.DMA,
        ],
    )
    return f(buckets.reshape(R_TOT, S), qkv_rows)


def _unsort_body(olzs_hbm, inv_hbm, olzu_hbm, inv_v, gidx_v, b0, b1, sem):
    wid = lax.axis_index("s") * 2 + lax.axis_index("c")

    def row_body(k, _):
        row = wid * _RPW + k
        base = row * S
        pltpu.sync_copy(inv_hbm.at[row], inv_v)
        for c in range(_GC):
            def gb(j, _, c=c):
                iv = inv_v[pl.ds(c * _GB + j * 16, 16)]
                gidx_v[c, pl.ds(j * 16, 16)] = iv + base
                return 0
            lax.fori_loop(0, _GB // 16, gb, 0)
        bufs = (b0, b1)
        waits = [None, None]
        for c in range(_GC):
            w = pltpu.async_copy(olzs_hbm.at[gidx_v.at[c]], bufs[c % 2], sem)
            if c > 0:
                waits[(c - 1) % 2].wait()
                pltpu.sync_copy(bufs[(c - 1) % 2],
                                olzu_hbm.at[pl.ds(base + (c - 1) * _GB, _GB)])
            waits[c % 2] = w
        waits[(_GC - 1) % 2].wait()
        pltpu.sync_copy(bufs[(_GC - 1) % 2],
                        olzu_hbm.at[pl.ds(base + (_GC - 1) * _GB, _GB)])
        return 0

    lax.fori_loop(0, _RPW, row_body, 0)


def _unsort(olz_s_rows, inv):
    f = pl.kernel(
        _unsort_body,
        out_type=jax.ShapeDtypeStruct((R_TOT * S, PK), jnp.float32),
        mesh=_sc_mesh(),
        compiler_params=_SC_PARAMS,
        scratch_types=[
            pltpu.VMEM((S,), jnp.int32),
            pltpu.VMEM((_GC, _GB), jnp.int32),
            pltpu.VMEM((_GB, PK), jnp.float32),
            pltpu.VMEM((_GB, PK), jnp.float32),
            pltpu.SemaphoreType.DMA,
        ],
    )
    return f(olz_s_rows, inv)


_QT = 256
_KW = _QT + BUCKET


def _attn_mask_np():
    q = np.arange(_QT)
    qc = q // BUCKET
    mask = np.full((_QT, _KW), -1e9, dtype=np.float32)
    for ci in range(4):
        mask[np.ix_(qc == ci, np.arange(ci * BUCKET, (ci + 2) * BUCKET))] = 0.0
    for i in range(_QT):
        ci = i // BUCKET
        mask[i, (ci + 1) * BUCKET + (i % BUCKET)] = -1e5
    return mask


def _attn_body(mask_ref, qkv_ref, o_ref):
    qkv = qkv_ref[0]
    qk = qkv[:, :D_HEAD]
    v = qkv[:, D_HEAD:]
    mask = mask_ref[...]
    inv_sqrt = 1.0 / math.sqrt(D_HEAD)
    for t in range(S // _QT):
        q = qk[t * _QT:(t + 1) * _QT, :]
        if t == 0:
            kwin = jnp.concatenate([qk[S - BUCKET:, :], qk[:_QT, :]], axis=0)
            vwin = jnp.concatenate([v[S - BUCKET:, :], v[:_QT, :]], axis=0)
        else:
            kwin = qk[t * _QT - BUCKET:(t + 1) * _QT, :]
            vwin = v[t * _QT - BUCKET:(t + 1) * _QT, :]
        s = lax.dot_general(q, kwin, (((1,), (1,)), ((), ())),
                            preferred_element_type=jnp.float32) * inv_sqrt
        s = s + mask
        m = jnp.max(s, axis=-1, keepdims=True)
        e = jnp.exp(s - m)
        z = jnp.sum(e, axis=-1, keepdims=True)
        o = jnp.dot(e / (z + 1e-9), vwin, preferred_element_type=jnp.float32)
        lz = m + jnp.log(z + 1e-9)
        o_ref[0, t * _QT:(t + 1) * _QT, :D_HEAD] = o
        o_ref[0, t * _QT:(t + 1) * _QT, D_HEAD:] = jnp.broadcast_to(lz, (_QT, D_HEAD))


def _attention(qkv_s, mask):
    return pl.pallas_call(
        _attn_body,
        grid=(R_TOT,),
        in_specs=[pl.BlockSpec((_QT, _KW), lambda r: (0, 0)),
                  pl.BlockSpec((1, S, PK), lambda r: (r, 0, 0))],
        out_specs=pl.BlockSpec((1, S, PK), lambda r: (r, 0, 0)),
        out_shape=jax.ShapeDtypeStruct((R_TOT, S, PK), jnp.float32),
    )(mask, qkv_s)


_MT = 256


def _merge_body(o_ref, h_ref, wo_ref, g_ref, be_ref, o_out):
    ob = o_ref[...]
    lz = ob[:, :, :, D_HEAD]
    m = jnp.max(lz, axis=0)
    w = jnp.exp(lz - m[None])
    den = jnp.sum(w, axis=0) + 1e-9
    cols = []
    for hd in range(N_HEADS):
        acc = jnp.zeros((_MT, D_HEAD), jnp.float32)
        for hh in range(N_HASHES):
            acc = acc + ob[hh, hd, :, :D_HEAD] * w[hh, hd][:, None]
        cols.append(acc / den[hd][:, None])
    attn = jnp.concatenate(cols, axis=-1)
    res = h_ref[...] + jnp.dot(attn, wo_ref[...], preferred_element_type=jnp.float32)
    mu = jnp.mean(res, axis=-1, keepdims=True)
    var = jnp.mean((res - mu) ** 2, axis=-1, keepdims=True)
    o_out[...] = (res - mu) / jnp.sqrt(var + 1e-5) * g_ref[...] + be_ref[...]


def _merge_outproj_ln(o_u, h2, Wo, g, beta):
    TPB = S // _MT
    return pl.pallas_call(
        _merge_body,
        grid=(B * S // _MT,),
        in_specs=[
            pl.BlockSpec((N_HASHES, N_HEADS, _MT, PK),
                         lambda i: (0, i // TPB, i % TPB, 0)),
            pl.BlockSpec((_MT, D_MODEL), lambda i: (i, 0)),
            pl.BlockSpec((D_MODEL, D_MODEL), lambda i: (0, 0)),
            pl.BlockSpec((D_MODEL,), lambda i: (0,)),
            pl.BlockSpec((D_MODEL,), lambda i: (0,)),
        ],
        out_specs=pl.BlockSpec((_MT, D_MODEL), lambda i: (i, 0)),
        out_shape=jax.ShapeDtypeStruct((B * S, D_MODEL), jnp.float32),
    )(o_u, h2, Wo, g, beta)


_FT = 256


def _ffn_body(h_ref, w1_ref, b1_ref, w2_ref, b2_ref, g_ref, be_ref, m_ref, o_ref,
              *, final):
    h = h_ref[...]
    f = jnp.dot(h, w1_ref[...], preferred_element_type=jnp.float32) + b1_ref[...]
    f = jax.nn.gelu(f)
    f = jnp.dot(f, w2_ref[...], preferred_element_type=jnp.float32) + b2_ref[...]
    res = h + f
    mu = jnp.mean(res, axis=-1, keepdims=True)
    var = jnp.mean((res - mu) ** 2, axis=-1, keepdims=True)
    out = (res - mu) / jnp.sqrt(var + 1e-5) * g_ref[...] + be_ref[...]
    if final:
        out = jax.nn.gelu(out) * m_ref[...]
    o_ref[...] = out


def _ffn_ln(h2, W1, b1, W2, b2, g, beta, mask2, final):
    return pl.pallas_call(
        functools.partial(_ffn_body, final=final),
        grid=(B * S // _FT,),
        in_specs=[pl.BlockSpec((_FT, D_MODEL), lambda i: (i, 0)),
                  pl.BlockSpec((D_MODEL, D_FF), lambda i: (0, 0)),
                  pl.BlockSpec((D_FF,), lambda i: (0,)),
                  pl.BlockSpec((D_FF, D_MODEL), lambda i: (0, 0)),
                  pl.BlockSpec((D_MODEL,), lambda i: (0,)),
                  pl.BlockSpec((D_MODEL,), lambda i: (0,)),
                  pl.BlockSpec((D_MODEL,), lambda i: (0,)),
                  pl.BlockSpec((_FT, 1), lambda i: (i, 0))],
        out_specs=pl.BlockSpec((_FT, D_MODEL), lambda i: (i, 0)),
        out_shape=jax.ShapeDtypeStruct((B * S, D_MODEL), jnp.float32),
    )(h2, W1, b1, W2, b2, g, beta, mask2)


_KB = 16384


def _head_body(x_ref, w_ref, b_ref, o_ref):
    k = pl.program_id(0)

    @pl.when(k == 0)
    def _init():
        o_ref[...] = jnp.broadcast_to(b_ref[...], o_ref.shape)

    o_ref[...] += jnp.dot(x_ref[...], w_ref[...], preferred_element_type=jnp.float32)


def _cls_head(flat, W_cls, b_cls):
    return pl.pallas_call(
        _head_body,
        grid=(S * D_MODEL // _KB,),
        in_specs=[pl.BlockSpec((B, _KB), lambda k: (0, k)),
                  pl.BlockSpec((_KB, NUM_CLASS), lambda k: (k, 0)),
                  pl.BlockSpec((NUM_CLASS,), lambda k: (0,))],
        out_specs=pl.BlockSpec((B, NUM_CLASS), lambda k: (0, 0)),
        out_shape=jax.ShapeDtypeStruct((B, NUM_CLASS), jnp.float32),
    )(flat, W_cls, b_cls)


def _layer(h2, mask2, Wqk, Wv, Wo, rot_cat, attn_mask,
           g1, beta1, W1, b1, W2, b2, g2, beta2, final):
    qkv, buckets = _proj_hash(h2, Wqk, Wv, rot_cat)
    qkv_rows = qkv.reshape(B * S * N_HEADS, PK)
    inv, qkvs = _sort_gather(buckets, qkv_rows)
    olz_s = _attention(qkvs.reshape(R_TOT, S, PK), attn_mask)
    olz_u = _unsort(olz_s.reshape(R_TOT * S, PK), inv)
    h2 = _merge_outproj_ln(olz_u.reshape(N_HASHES, NROW, S, PK),
                           h2, Wo, g1, beta1)
    return _ffn_ln(h2, W1, b1, W2, b2, g2, beta2, mask2, final)


def kernel(x, mask, W_embed, rotations, W_cls, b_cls,
           Wqk_0, Wv_0, Wo_0, W1_0, b1_0, W2_0, b2_0, g1_0, beta1_0, g2_0, beta2_0,
           Wqk_1, Wv_1, Wo_1, W1_1, b1_1, W2_1, b2_1, g1_1, beta1_1, g2_1, beta2_1):
    rot_cat = rotations.transpose(1, 0, 2).reshape(D_HEAD, N_HASHES * BUCKET)
    attn_mask = jnp.asarray(_attn_mask_np())
    mask2 = mask.reshape(B * S, 1)
    h2 = _embed(x, W_embed)
    h2 = _layer(h2, mask2, Wqk_0, Wv_0, Wo_0, rot_cat, attn_mask,
                g1_0, beta1_0, W1_0, b1_0, W2_0, b2_0, g2_0, beta2_0, False)
    h2 = _layer(h2, mask2, Wqk_1, Wv_1, Wo_1, rot_cat, attn_mask,
                g1_1, beta1_1, W1_1, b1_1, W2_1, b2_1, g2_1, beta2_1, True)
    flat = h2.reshape(B, S * D_MODEL)
    return _cls_head(flat, W_cls, b_cls)
